# baseline (speedup 1.0000x reference)
import functools

import jax
import jax.numpy as jnp
from jax import lax
from jax.experimental import pallas as pl
from jax.experimental.pallas import tpu as pltpu
from jax.experimental.pallas import tpu_sc as plsc

_WIDTHS = (2, 2, 1, 6, 18, 18, 12, 12, 12, 18)
_NUM_TABLES = len(_WIDTHS)


def _body(x_hbm, *refs):
    ws = refs[:_NUM_TABLES]
    outs = refs[_NUM_TABLES:2 * _NUM_TABLES]
    x_s = refs[2 * _NUM_TABLES]
    sem = refs[2 * _NUM_TABLES + 1]

    @pl.when(lax.axis_index("q") == 0)
    def _():
        pltpu.sync_copy(x_hbm, x_s)
        v = x_s[0, 0] * 100.0
        i0 = v.astype(jnp.int32)
        idx = jnp.where(i0.astype(jnp.float32) > v, i0 - 1, i0)
        copies = [pltpu.async_copy(w.at[pl.ds(idx, 1), :], o, sem)
                  for w, o in zip(ws, outs)]
        for c in copies:
            c.wait()


_scs = functools.partial(
    pl.kernel,
    out_type=[jax.ShapeDtypeStruct((1, d), jnp.float32) for d in _WIDTHS],
    mesh=plsc.ScalarSubcoreMesh(axis_name="q", num_cores=1),
    scratch_types=[
        pltpu.SMEM((1, 1), jnp.float32),
        pltpu.SemaphoreType.DMA,
    ],
    compiler_params=pltpu.CompilerParams(needs_layout_passes=False,
                                         use_tc_tiling_on_sc=False),
)(_body)


def kernel(x, W_enc_embed, W_dec_embed, W_enc_layer, W_dec_layer,
           W_enc_ffn, W_dec_ffn, W_enc_heads, W_dec_heads,
           W_dec_ende_heads, W_dec_arb_ende):
    (enc_embed, dec_embed, enc_layer, dec_layer, enc_ffn, dec_ffn,
     enc_heads, dec_heads, dec_ende_heads, dec_arb_ende) = _scs(
        x, W_enc_embed, W_dec_embed, W_enc_layer, W_dec_layer,
        W_enc_ffn, W_dec_ffn, W_enc_heads, W_dec_heads,
        W_dec_ende_heads, W_dec_arb_ende)
    return (enc_embed, dec_embed, enc_layer.reshape(1, 1),
            dec_layer.reshape(1, 6),
            enc_ffn.reshape(6, 3), dec_ffn.reshape(6, 3),
            enc_heads.reshape(6, 2), dec_heads.reshape(6, 2),
            dec_ende_heads.reshape(6, 2), dec_arb_ende.reshape(6, 3))
